# Initial kernel scaffold; baseline (speedup 1.0000x reference)
import jax
import jax.numpy as jnp
from jax.experimental import pallas as pl


def kernel(x, edge_index, edge_type, W_rel1, W_root1, b1, W_rel2, W_root2, b2):
    raise NotImplementedError("write your pallas kernel here")


# speedup vs baseline: 25.6661x; 25.6661x over previous
import functools

import jax
import jax.numpy as jnp
from jax import lax
from jax.experimental import pallas as pl
from jax.experimental.pallas import tpu as pltpu
from jax.experimental.pallas import tpu_sc as plsc

N = 10000
E = 320000
D = 128
R = 8
NR = N * R

NC = 2
NS = 16
NW = NC * NS
L = 16

KE = 80
EPT = E // NW
EPC = E // NS
NB = EPT // KE
NBA = EPC // KE
W = 3
WA = 5
NBM = (NB // W) * W
CH = 640

_mesh = plsc.VectorSubcoreMesh(
    core_axis_name="c", subcore_axis_name="s", num_cores=NC, num_subcores=NS)
_sc_params = pltpu.CompilerParams(use_tc_tiling_on_sc=False)


@functools.partial(
    pl.kernel,
    mesh=_mesh,
    compiler_params=_sc_params,
    out_type=jax.ShapeDtypeStruct((NR, L), jnp.float32),
    scratch_types=[
        pltpu.MemorySpace.VMEM_SHARED((NR, L), jnp.float32),
        [pltpu.VMEM((KE,), jnp.int32) for _ in range(WA)],
        pltpu.VMEM((KE, L), jnp.float32),
        pltpu.VMEM((CH, L), jnp.float32),
        pltpu.SemaphoreType.DMA((WA,)),
        pltpu.SemaphoreType.DMA((WA,)),
    ],
)
def _sc_prep(packed_hbm, ones_hbm, zcnt_hbm,
             norm_hbm,
             cnt_sh, compb, onesb, chunk, sem_i, sem_c):
    cid = lax.axis_index("c")
    sid = lax.axis_index("s")
    wid = cid * NS + sid

    pltpu.sync_copy(zcnt_hbm, cnt_sh.at[pl.ds(sid * (NR // NS), NR // NS)])
    pltpu.sync_copy(ones_hbm, onesb)
    plsc.subcore_barrier()

    def body_a(j, _):
        @pl.when(j > 0)
        def _():
            for u in range(WA):
                pltpu.make_async_copy(
                    onesb, cnt_sh.at[compb[u]], sem_c.at[u]).wait()
        cps = [
            pltpu.async_copy(packed_hbm.at[sid * NBA + j * WA + u, 2],
                             compb[u], sem_i.at[u])
            for u in range(WA)
        ]
        for u in range(WA):
            cps[u].wait()
            pltpu.async_copy(onesb, cnt_sh.at[compb[u]], sem_c.at[u],
                             add=True)
        return ()

    lax.fori_loop(0, NBA // WA, body_a, ())
    for u in range(WA):
        pltpu.make_async_copy(onesb, cnt_sh.at[compb[u]], sem_c.at[u]).wait()
    plsc.subcore_barrier()

    def body_b(c, _):
        @pl.when(lax.rem(c, NW) == wid)
        def _():
            base = c * CH
            pltpu.sync_copy(cnt_sh.at[pl.ds(base, CH)], chunk)

            @plsc.parallel_loop(0, CH, step=1, unroll=8)
            def _(r):
                v = chunk[r, pl.ds(0, L)]
                chunk[r, pl.ds(0, L)] = 1.0 / jnp.maximum(v, 1.0)

            pltpu.sync_copy(chunk, norm_hbm.at[pl.ds(base, CH)])
        return ()

    lax.fori_loop(0, NR // CH, body_b, ())


@functools.partial(
    pl.kernel,
    mesh=_mesh,
    compiler_params=_sc_params,
    out_type=jax.ShapeDtypeStruct((NC, N, D), jnp.float32),
    scratch_types=[
        pltpu.MemorySpace.VMEM_SHARED((N, D), jnp.float32),
        [pltpu.VMEM((3, KE), jnp.int32) for _ in range(W)],
        [pltpu.VMEM((KE, L), jnp.float32) for _ in range(W)],
        [pltpu.VMEM((KE, D // 2), jnp.int32) for _ in range(W)],
        [pltpu.VMEM((KE, D), jnp.float32) for _ in range(W)],
        pltpu.SemaphoreType.DMA((W,)),
        pltpu.SemaphoreType.DMA((W,)),
        pltpu.SemaphoreType.DMA((W,)),
        pltpu.SemaphoreType.DMA((W,)),
    ],
)
def _sc_edge(h_hbm, packed_hbm, norm_hbm, zacc_hbm,
             out_hbm,
             acc_sh, ibuf, nrows, brows, rows, sem_i, sem_h, sem_n, sem_s):
    cid = lax.axis_index("c")
    sid = lax.axis_index("s")
    wid = cid * NS + sid
    nps = N // NS

    pltpu.sync_copy(zacc_hbm, acc_sh.at[pl.ds(sid * nps, nps)])
    plsc.subcore_barrier()

    def body(j, _):
        @pl.when(j > 0)
        def _():
            for u in range(W):
                pltpu.make_async_copy(
                    rows[u], acc_sh.at[ibuf[u].at[1]], sem_s.at[u]).wait()
        gb = wid * NB + j * W
        cps_i = [
            pltpu.async_copy(packed_hbm.at[gb + u], ibuf[u], sem_i.at[u])
            for u in range(W)
        ]
        cps = []
        for u in range(W):
            cps_i[u].wait()
            cps.append((
                pltpu.async_copy(h_hbm.at[ibuf[u].at[0]], brows[u],
                                 sem_h.at[u]),
                pltpu.async_copy(norm_hbm.at[ibuf[u].at[2]], nrows[u],
                                 sem_n.at[u]),
            ))
        for u in range(W):
            cp_h, cp_n = cps[u]
            cp_n.wait()
            cp_h.wait()
            nrows_u = nrows[u]
            brows_u = brows[u]
            rows_u = rows[u]

            @plsc.parallel_loop(0, KE, step=1, unroll=8)
            def _(e):
                spl = nrows_u[e, pl.ds(0, L)]
                for g in range(D // (2 * L)):
                    w32 = brows_u[e, pl.ds(g * L, L)]
                    a = lax.bitcast_convert_type(w32 << 16, jnp.float32)
                    b = lax.bitcast_convert_type(w32 & jnp.int32(-65536), jnp.float32)
                    rows_u[e, pl.ds(g * L, L)] = a * spl
                    rows_u[e, pl.ds(D // 2 + g * L, L)] = b * spl

            pltpu.async_copy(
                rows[u], acc_sh.at[ibuf[u].at[1]], sem_s.at[u], add=True)
        return ()

    lax.fori_loop(0, NB // W, body, ())
    for u in range(W):
        pltpu.make_async_copy(
            rows[u], acc_sh.at[ibuf[u].at[1]], sem_s.at[u]).wait()
    for t in range(NB - NBM):
        pltpu.async_copy(packed_hbm.at[wid * NB + NBM + t], ibuf[0],
                         sem_i.at[0]).wait()
        cp_h = pltpu.async_copy(h_hbm.at[ibuf[0].at[0]], brows[0], sem_h.at[0])
        cp_n = pltpu.async_copy(norm_hbm.at[ibuf[0].at[2]], nrows[0],
                                sem_n.at[0])
        cp_n.wait()
        cp_h.wait()
        nrows_t = nrows[0]
        brows_t = brows[0]
        rows_t = rows[0]

        @plsc.parallel_loop(0, KE, step=1, unroll=8)
        def _(e):
            spl = nrows_t[e, pl.ds(0, L)]
            for g in range(D // (2 * L)):
                w32 = brows_t[e, pl.ds(g * L, L)]
                a = lax.bitcast_convert_type(w32 << 16, jnp.float32)
                b = lax.bitcast_convert_type(w32 & jnp.int32(-65536), jnp.float32)
                rows_t[e, pl.ds(g * L, L)] = a * spl
                rows_t[e, pl.ds(D // 2 + g * L, L)] = b * spl

        pltpu.async_copy(
            rows[0], acc_sh.at[ibuf[0].at[1]], sem_s.at[0], add=True).wait()
    plsc.subcore_barrier()
    pltpu.sync_copy(acc_sh.at[pl.ds(sid * nps, nps)],
                    out_hbm.at[cid, pl.ds(sid * nps, nps)])


BN = 2000


def _interleave_bf16(v):
    def rnd(x):
        bits = lax.bitcast_convert_type(x, jnp.int32)
        return bits + 0x7FFF + ((bits >> 16) & 1)

    a = rnd(v[:, : D // 2])
    b = rnd(v[:, D // 2:])
    return ((a >> 16) & 0xFFFF) | (b & jnp.int32(-65536))


def _mm_rel_body(x_ref, w_ref, o_ref):
    o_ref[0] = _interleave_bf16(
        jnp.dot(x_ref[...], w_ref[0], preferred_element_type=jnp.float32))


def _mm_rel(x, w_rel):
    return pl.pallas_call(
        _mm_rel_body,
        grid=(R, N // BN),
        in_specs=[
            pl.BlockSpec((BN, D), lambda r, i: (i, 0)),
            pl.BlockSpec((1, D, D), lambda r, i: (r, 0, 0)),
        ],
        out_specs=pl.BlockSpec((1, BN, D // 2), lambda r, i: (r, i, 0)),
        out_shape=jax.ShapeDtypeStruct((R, N, D // 2), jnp.int32),
    )(x, w_rel)


def _comb_mm_body(p_ref, x_ref, wroot_ref, b_ref, wrel_ref, z_ref, h_ref):
    r = pl.program_id(1)

    @pl.when(r == 0)
    def _():
        z_ref[...] = jnp.maximum(
            p_ref[0] + p_ref[1] + b_ref[...]
            + jnp.dot(x_ref[...], wroot_ref[...],
                      preferred_element_type=jnp.float32), 0.0)

    h_ref[0] = _interleave_bf16(
        jnp.dot(z_ref[...], wrel_ref[0], preferred_element_type=jnp.float32))


def _comb_mm(parts, x, w_root, b, w_rel):
    return pl.pallas_call(
        _comb_mm_body,
        grid=(N // BN, R),
        in_specs=[
            pl.BlockSpec((2, BN, D), lambda i, r: (0, i, 0)),
            pl.BlockSpec((BN, D), lambda i, r: (i, 0)),
            pl.BlockSpec((D, D), lambda i, r: (0, 0)),
            pl.BlockSpec((1, D), lambda i, r: (0, 0)),
            pl.BlockSpec((1, D, D), lambda i, r: (r, 0, 0)),
        ],
        out_specs=(
            pl.BlockSpec((BN, D), lambda i, r: (i, 0)),
            pl.BlockSpec((1, BN, D // 2), lambda i, r: (r, i, 0)),
        ),
        out_shape=(
            jax.ShapeDtypeStruct((N, D), jnp.float32),
            jax.ShapeDtypeStruct((R, N, D // 2), jnp.int32),
        ),
    )(parts, x, w_root, b, w_rel)


def _comb_body(p_ref, x_ref, w_ref, b_ref, o_ref, *, act):
    v = (p_ref[0] + p_ref[1] + b_ref[...]
         + jnp.dot(x_ref[...], w_ref[...], preferred_element_type=jnp.float32))
    o_ref[...] = jnp.maximum(v, 0.0) if act else v


def _combine(parts, x, w_root, b, act):
    return pl.pallas_call(
        functools.partial(_comb_body, act=act),
        grid=(N // BN,),
        in_specs=[
            pl.BlockSpec((2, BN, D), lambda i: (0, i, 0)),
            pl.BlockSpec((BN, D), lambda i: (i, 0)),
            pl.BlockSpec((D, D), lambda i: (0, 0)),
            pl.BlockSpec((1, D), lambda i: (0, 0)),
        ],
        out_specs=pl.BlockSpec((BN, D), lambda i: (i, 0)),
        out_shape=jax.ShapeDtypeStruct((N, D), jnp.float32),
    )(parts, x, w_root, b)


def kernel(x, edge_index, edge_type, W_rel1, W_root1, b1, W_rel2, W_root2, b2):
    src = edge_index[0]
    dst = edge_index[1]
    comp = dst * R + edge_type
    gidx = edge_type * N + src
    packed = (jnp.stack([gidx, dst, comp], 0)
              .reshape(3, NW * NB, KE).transpose(1, 0, 2))
    ones = jnp.ones((KE, L), jnp.float32)
    zcnt = jnp.zeros((NR // NS, L), jnp.float32)
    zacc = jnp.zeros((N // NS, D), jnp.float32)

    norm = _sc_prep(packed, ones, zcnt)

    h1 = _mm_rel(x, W_rel1).reshape(R * N, D // 2)
    parts1 = _sc_edge(h1, packed, norm, zacc)
    z, h2 = _comb_mm(parts1, x, W_root1, b1.reshape(1, D), W_rel2)

    parts2 = _sc_edge(h2.reshape(R * N, D // 2), packed, norm, zacc)
    out = _combine(parts2, z, W_root2, b2.reshape(1, D), act=False)
    return out
